# Initial kernel scaffold; baseline (speedup 1.0000x reference)
import jax
import jax.numpy as jnp
from jax.experimental import pallas as pl


def kernel(x, edge_index, batch, xcent_base, W_l, b_l, W_r, b_r, att, bias):
    raise NotImplementedError("write your pallas kernel here")


# speedup vs baseline: 143.6424x; 143.6424x over previous
import jax
import jax.numpy as jnp
from jax import lax
from jax.experimental import pallas as pl
from jax.experimental.pallas import tpu as pltpu

IN_C = 128
HEADS = 2
RATIO = 16
NBATCH = 8
NDST = NBATCH * RATIO
NEG_SLOPE = 0.2
TILE_N = 1024


def _bipart_pool_kernel(x_ref, batch_ref, xcb_ref, wl_ref, bl_ref, wr_ref,
                        br_ref, att_ref, bias_ref, out_ref,
                        xl_s, logc_s, mseg_s, num_s, den_s):
    f32 = jnp.float32
    phase = pl.program_id(0)
    t = pl.program_id(1)
    ntiles = pl.num_programs(1)
    rows = pl.ds(t * TILE_N, TILE_N)

    lane_r = lax.broadcasted_iota(jnp.int32, (1, RATIO), 1)

    @pl.when(phase == 0)
    def _phase0():
        xs = x_ref[rows, :]
        batch_t = batch_ref[rows, :]
        xl_t = jnp.dot(xs, wl_ref[...], preferred_element_type=f32) + bl_ref[...]
        xl_s[rows, :] = xl_t
        xr = (jnp.dot(xcb_ref[...], wr_ref[...], preferred_element_type=f32)
              + br_ref[...])
        for h in range(HEADS):
            xl_h = xl_t[:, h * IN_C:(h + 1) * IN_C]
            att_h = att_ref[h:h + 1, :]
            logc = jnp.zeros((TILE_N, RATIO), f32)
            for r in range(RATIO):
                z = xl_h + xr[r:r + 1, h * IN_C:(h + 1) * IN_C]
                lrelu = jnp.where(z > 0, z, NEG_SLOPE * z)
                lr = jnp.sum(lrelu * att_h, axis=1, keepdims=True)
                logc = jnp.where(lane_r == r, lr, logc)
            logc_s[rows, h * RATIO:(h + 1) * RATIO] = logc
            mrows = [jnp.max(jnp.where(batch_t == b, logc, -1e30),
                             axis=0, keepdims=True) for b in range(NBATCH)]
            m_t = jnp.concatenate(mrows, axis=0)
            sl = (slice(0, NBATCH), slice(h * RATIO, (h + 1) * RATIO))

            @pl.when(t == 0)
            def _():
                mseg_s[sl] = m_t

            @pl.when(t > 0)
            def _():
                mseg_s[sl] = jnp.maximum(mseg_s[sl], m_t)

    @pl.when(phase == 1)
    def _phase1():
        batch_t = batch_ref[rows, :]
        valid = batch_t < NBATCH
        onehot = (batch_t == lax.broadcasted_iota(jnp.int32, (1, NBATCH), 1)
                  ).astype(f32)
        til = (lax.broadcasted_iota(jnp.int32, (RATIO, NDST), 1) % RATIO ==
               lax.broadcasted_iota(jnp.int32, (RATIO, NDST), 0)).astype(f32)
        rep = (lax.broadcasted_iota(jnp.int32, (NBATCH, NDST), 1) // RATIO ==
               lax.broadcasted_iota(jnp.int32, (NBATCH, NDST), 0)).astype(f32)
        oh_rep = jnp.dot(onehot, rep, preferred_element_type=f32)
        for h in range(HEADS):
            xl_h = xl_s[rows, h * IN_C:(h + 1) * IN_C]
            logc = logc_s[rows, h * RATIO:(h + 1) * RATIO]
            mseg = mseg_s[0:NBATCH, h * RATIO:(h + 1) * RATIO]
            mrow = jnp.dot(onehot, mseg, preferred_element_type=f32)
            pc = jnp.exp(jnp.where(valid, logc - mrow, -1e30))
            pfull = jnp.dot(pc, til, preferred_element_type=f32) * oh_rep
            num_t = lax.dot_general(pfull, xl_h, (((0,), (0,)), ((), ())),
                                    preferred_element_type=f32)
            den_t = jnp.sum(pfull, axis=0, keepdims=True)
            nsl = (slice(0, NDST), slice(h * IN_C, (h + 1) * IN_C))
            dsl = (slice(h, h + 1), slice(0, NDST))

            @pl.when(t == 0)
            def _():
                num_s[nsl] = num_t
                den_s[dsl] = den_t

            @pl.when(t > 0)
            def _():
                num_s[nsl] = num_s[nsl] + num_t
                den_s[dsl] = den_s[dsl] + den_t

        @pl.when(t == ntiles - 1)
        def _finalize():
            acc = jnp.zeros((NDST, IN_C), f32)
            for h in range(HEADS):
                den_col = jnp.transpose(den_s[h:h + 1, 0:NDST])
                acc = acc + (num_s[0:NDST, h * IN_C:(h + 1) * IN_C]
                             / (den_col + 1e-16))
            out_ref[...] = acc * (1.0 / HEADS) + bias_ref[...]


def kernel(x, edge_index, batch, xcent_base, W_l, b_l, W_r, b_r, att, bias):
    del edge_index
    n = x.shape[0]
    n_pad = -(-n // TILE_N) * TILE_N
    ntiles = n_pad // TILE_N
    xp = jnp.pad(x, ((0, n_pad - n), (0, 0)))
    bp = jnp.pad(batch.astype(jnp.int32), (0, n_pad - n),
                 constant_values=NBATCH).reshape(n_pad, 1)
    out = pl.pallas_call(
        _bipart_pool_kernel,
        grid=(2, ntiles),
        out_shape=jax.ShapeDtypeStruct((NDST, IN_C), jnp.float32),
        scratch_shapes=[
            pltpu.VMEM((n_pad, HEADS * IN_C), jnp.float32),
            pltpu.VMEM((n_pad, HEADS * RATIO), jnp.float32),
            pltpu.VMEM((NBATCH, HEADS * RATIO), jnp.float32),
            pltpu.VMEM((NDST, HEADS * IN_C), jnp.float32),
            pltpu.VMEM((HEADS, NDST), jnp.float32),
        ],
    )(xp, bp, xcent_base, W_l, b_l.reshape(1, HEADS * IN_C), W_r,
      b_r.reshape(1, HEADS * IN_C), att, bias.reshape(1, IN_C))
    return out.reshape(NBATCH, RATIO, IN_C)


# speedup vs baseline: 244.3835x; 1.7013x over previous
import jax
import jax.numpy as jnp
from jax import lax
from jax.experimental import pallas as pl
from jax.experimental.pallas import tpu as pltpu

IN_C = 128
HEADS = 2
RATIO = 16
NBATCH = 8
NDST = NBATCH * RATIO
NEG_SLOPE = 0.2
TILE_N = 1024
MASKNEG = -3e38
MFLOOR = -1e33


def _bipart_pool_kernel(x_ref, batch_ref, xcb_ref, wl_ref, bl_ref, wr_ref,
                        br_ref, att_ref, attT_ref, biasT_ref, out_ref,
                        m_s, den_s, numT_s):
    f32 = jnp.float32
    t = pl.program_id(0)
    ntiles = pl.num_programs(0)
    rows = pl.ds(t * TILE_N, TILE_N)

    lane_r = lax.broadcasted_iota(jnp.int32, (1, RATIO), 1)

    xs = x_ref[rows, :]
    batch_t = batch_ref[rows, :]
    xl = jnp.dot(xs, wl_ref[...], preferred_element_type=f32) + bl_ref[...]
    xr = (jnp.dot(xcb_ref[...], wr_ref[...], preferred_element_type=f32)
          + br_ref[...])

    onehot = (batch_t == lax.broadcasted_iota(jnp.int32, (1, NBATCH), 1)
              ).astype(f32)
    til = (lax.broadcasted_iota(jnp.int32, (RATIO, NDST), 1) % RATIO ==
           lax.broadcasted_iota(jnp.int32, (RATIO, NDST), 0)).astype(f32)
    rep = (lax.broadcasted_iota(jnp.int32, (NBATCH, NDST), 1) // RATIO ==
           lax.broadcasted_iota(jnp.int32, (NBATCH, NDST), 0)).astype(f32)
    oh_rep = jnp.dot(onehot, rep, preferred_element_type=f32)

    for h in range(HEADS):
        xl_h = xl[:, h * IN_C:(h + 1) * IN_C]
        att_col = attT_ref[:, h:h + 1]
        logc = jnp.zeros((TILE_N, RATIO), f32)
        for r in range(RATIO):
            z = xl_h + xr[r:r + 1, h * IN_C:(h + 1) * IN_C]
            t_r = jnp.maximum(z, NEG_SLOPE * z)
            s_r = jnp.where(lane_r == r, att_col, 0.0)
            logc = logc + jnp.dot(t_r, s_r, preferred_element_type=f32)
        l128 = (jnp.dot(logc, til, preferred_element_type=f32) * oh_rep
                + (oh_rep - 1.0) * (-MASKNEG))
        m_t = jnp.maximum(jnp.max(l128, axis=0, keepdims=True), MFLOOR)

        msl = (slice(h, h + 1), slice(0, NDST))
        nsl = (slice(0, IN_C), slice(h * NDST, (h + 1) * NDST))

        @pl.when(t == 0)
        def _():
            p = jnp.exp(l128 - m_t)
            m_s[msl] = m_t
            den_s[msl] = jnp.sum(p, axis=0, keepdims=True)
            numT_s[nsl] = lax.dot_general(xl_h, p, (((0,), (0,)), ((), ())),
                                          preferred_element_type=f32)

        @pl.when(t > 0)
        def _():
            m_old = m_s[msl]
            m_new = jnp.maximum(m_old, m_t)
            corr = jnp.exp(m_old - m_new)
            p = jnp.exp(l128 - m_new)
            m_s[msl] = m_new
            den_s[msl] = den_s[msl] * corr + jnp.sum(p, axis=0, keepdims=True)
            numT_s[nsl] = (numT_s[nsl] * corr
                           + lax.dot_general(xl_h, p, (((0,), (0,)), ((), ())),
                                             preferred_element_type=f32))

    @pl.when(t == ntiles - 1)
    def _finalize():
        acc = jnp.zeros((IN_C, NDST), f32)
        for h in range(HEADS):
            acc = acc + (numT_s[0:IN_C, h * NDST:(h + 1) * NDST]
                         / (den_s[h:h + 1, 0:NDST] + 1e-16))
        out_ref[...] = acc * (1.0 / HEADS) + biasT_ref[...]


def kernel(x, edge_index, batch, xcent_base, W_l, b_l, W_r, b_r, att, bias):
    del edge_index
    n = x.shape[0]
    n_pad = -(-n // TILE_N) * TILE_N
    ntiles = n_pad // TILE_N
    xp = jnp.pad(x, ((0, n_pad - n), (0, 0)))
    bp = jnp.pad(batch.astype(jnp.int32), (0, n_pad - n),
                 constant_values=NBATCH).reshape(n_pad, 1)
    outT = pl.pallas_call(
        _bipart_pool_kernel,
        grid=(ntiles,),
        out_shape=jax.ShapeDtypeStruct((IN_C, NDST), jnp.float32),
        scratch_shapes=[
            pltpu.VMEM((HEADS, NDST), jnp.float32),
            pltpu.VMEM((HEADS, NDST), jnp.float32),
            pltpu.VMEM((IN_C, HEADS * NDST), jnp.float32),
        ],
    )(xp, bp, xcent_base, W_l, b_l.reshape(1, HEADS * IN_C), W_r,
      b_r.reshape(1, HEADS * IN_C), att, att.T, bias.reshape(IN_C, 1))
    return outT.T.reshape(NBATCH, RATIO, IN_C)
